# baseline (speedup 1.0000x reference)
import jax
import jax.numpy as jnp
from jax import lax
from jax.experimental import pallas as pl
from jax.experimental.pallas import tpu as pltpu
from jax.experimental.pallas import tpu_sc as plsc

_INFO = plsc.get_sparse_core_info()
_NC = _INFO.num_cores
_NS = _INFO.num_subcores
_NW = _NC * _NS


def _copy_rows(src, dst, bufs, sin, sout, off, s):
    ch = bufs[0].shape[0]
    chunks = []
    done = 0
    while done < s:
        c = min(ch, s - done)
        chunks.append((done, c))
        done += c

    def in_copy(i):
        o, c = chunks[i]
        return pltpu.make_async_copy(
            src.at[pl.ds(off + o, c), :], bufs[i % 2].at[pl.ds(0, c), :], sin[i % 2])

    def out_copy(i):
        o, c = chunks[i]
        return pltpu.make_async_copy(
            bufs[i % 2].at[pl.ds(0, c), :], dst.at[pl.ds(off + o, c), :], sout[i % 2])

    nck = len(chunks)
    in_copy(0).start()
    for i in range(nck):
        in_copy(i).wait()
        out_copy(i).start()
        if i + 1 < nck:
            if i >= 1:
                out_copy(i - 1).wait()
            in_copy(i + 1).start()
    for i in range(max(0, nck - 2), nck):
        out_copy(i).wait()


def _sc_copy_body(src, out, buf0, buf1, si0, si1, so0, so1):
    wid = lax.axis_index("s") * _NC + lax.axis_index("c")
    n = src.shape[0]
    base, rem = divmod(n // 8, _NW)
    rows_big = (base + 1) * 8
    rows_small = base * 8
    bufs = (buf0, buf1)
    sin = (si0, si1)
    sout = (so0, so1)
    if rem:
        @pl.when(wid < rem)
        def _():
            _copy_rows(src, out, bufs, sin, sout, wid * rows_big, rows_big)

        @pl.when(wid >= rem)
        def _():
            off = rem * rows_big + (wid - rem) * rows_small
            _copy_rows(src, out, bufs, sin, sout, off, rows_small)
    else:
        _copy_rows(src, out, bufs, sin, sout, wid * rows_small, rows_small)


def kernel(variable_features, constraint_features, edge_indices, reversed_edge_indices):
    n, d = variable_features.shape
    ch = (240_000 // (d * 4)) // 8 * 8
    mesh = plsc.VectorSubcoreMesh(core_axis_name="c", subcore_axis_name="s")
    out = pl.kernel(
        _sc_copy_body,
        out_type=jax.ShapeDtypeStruct((n, d), variable_features.dtype),
        mesh=mesh,
        scratch_types=[
            pltpu.VMEM((ch, d), jnp.float32),
            pltpu.VMEM((ch, d), jnp.float32),
            pltpu.SemaphoreType.DMA,
            pltpu.SemaphoreType.DMA,
            pltpu.SemaphoreType.DMA,
            pltpu.SemaphoreType.DMA,
        ],
        compiler_params=pltpu.CompilerParams(use_tc_tiling_on_sc=True),
    )(variable_features)
    return out


# speedup vs baseline: 1.0435x; 1.0435x over previous
import jax
import jax.numpy as jnp
from jax import lax
from jax.experimental import pallas as pl
from jax.experimental.pallas import tpu as pltpu
from jax.experimental.pallas import tpu_sc as plsc

_INFO = plsc.get_sparse_core_info()
_NC = _INFO.num_cores
_NS = _INFO.num_subcores
_NW = _NC * _NS


def _copy_rows(src, dst, buf, off, s):
    ch = buf.shape[0]
    done = 0
    while done < s:
        c = min(ch, s - done)
        pltpu.sync_copy(src.at[pl.ds(off + done, c), :], buf.at[pl.ds(0, c), :])
        pltpu.sync_copy(buf.at[pl.ds(0, c), :], dst.at[pl.ds(off + done, c), :])
        done += c


def _sc_copy_body(src, out, buf):
    wid = lax.axis_index("s") * _NC + lax.axis_index("c")
    n = src.shape[0]
    base, rem = divmod(n // 8, _NW)
    rows_big = (base + 1) * 8
    rows_small = base * 8
    if rem:
        @pl.when(wid < rem)
        def _():
            _copy_rows(src, out, buf, wid * rows_big, rows_big)

        @pl.when(wid >= rem)
        def _():
            off = rem * rows_big + (wid - rem) * rows_small
            _copy_rows(src, out, buf, off, rows_small)
    else:
        _copy_rows(src, out, buf, wid * rows_small, rows_small)


def kernel(variable_features, constraint_features, edge_indices, reversed_edge_indices):
    n, d = variable_features.shape
    ch = (480_000 // (d * 4)) // 8 * 8
    mesh = plsc.VectorSubcoreMesh(core_axis_name="c", subcore_axis_name="s")
    out = pl.kernel(
        _sc_copy_body,
        out_type=jax.ShapeDtypeStruct((n, d), variable_features.dtype),
        mesh=mesh,
        scratch_types=[pltpu.VMEM((ch, d), jnp.float32)],
        compiler_params=pltpu.CompilerParams(use_tc_tiling_on_sc=True,
                                             skip_device_barrier=True),
    )(variable_features)
    return out
